# Initial kernel scaffold; baseline (speedup 1.0000x reference)
import jax
import jax.numpy as jnp
from jax.experimental import pallas as pl


def kernel(x, edge_index, W1, b1, W2, b2):
    raise NotImplementedError("write your pallas kernel here")


# speedup vs baseline: 8.6733x; 8.6733x over previous
import functools

import jax
import jax.numpy as jnp
from jax import lax
from jax.experimental import pallas as pl
from jax.experimental.pallas import tpu as pltpu
from jax.experimental.pallas import tpu_sc as plsc

N_NODES = 10000
D_FEAT = 128
N_EDGES = 320000
HIDDEN = 128

NC = 2
NS = 16
NW = NC * NS
LANES = 16

CHUNK = 128
CHUNKS_PER_W = 80
EDGES_PER_W = CHUNK * CHUNKS_PER_W
E_PAD = EDGES_PER_W * NW
ACC_ROWS = 10240
ROWS_PER_SUB = ACC_ROWS // NS
ZCHUNKS = ROWS_PER_SUB // CHUNK
TRASH = N_NODES

_mesh = plsc.VectorSubcoreMesh(core_axis_name="c", subcore_axis_name="s")


def _fill(ref, value):
    rows, cols = ref.shape

    @pl.loop(0, rows)
    def _(i):
        @pl.loop(0, cols, step=LANES)
        def _(j):
            ref[i, pl.ds(j, LANES)] = jnp.full((LANES,), value, jnp.float32)


def _sc_degree(dst_hbm):
    @functools.partial(
        pl.kernel,
        out_type=jax.ShapeDtypeStruct((NC, ACC_ROWS, LANES), jnp.float32),
        mesh=_mesh,
        scratch_types=[
            pltpu.VMEM((CHUNKS_PER_W, CHUNK), jnp.int32),
            pltpu.VMEM((CHUNK, LANES), jnp.float32),
            pltpu.VMEM_SHARED((ACC_ROWS, LANES), jnp.float32),
        ],
    )
    def k(dst_h, out_h, dst_v, const_v, acc):
        c = lax.axis_index("c")
        s = lax.axis_index("s")
        wid = c * NS + s
        base = s * ROWS_PER_SUB

        pltpu.sync_copy(dst_h.at[wid], dst_v)

        _fill(const_v, 0.0)

        @pl.loop(0, ZCHUNKS)
        def _(i):
            pltpu.sync_copy(const_v, acc.at[pl.ds(base + i * CHUNK, CHUNK)])

        _fill(const_v, 1.0)
        plsc.subcore_barrier()

        @pl.loop(0, CHUNKS_PER_W)
        def _(j):
            pltpu.sync_copy(const_v, acc.at[dst_v.at[j]], add=True)

        plsc.subcore_barrier()
        pltpu.sync_copy(acc.at[pl.ds(base, ROWS_PER_SUB)],
                        out_h.at[c, pl.ds(base, ROWS_PER_SUB)])

    return k(dst_hbm)


def _sc_segment_sum(g, src_hbm, dst_hbm):
    @functools.partial(
        pl.kernel,
        out_type=jax.ShapeDtypeStruct((NC, ACC_ROWS, HIDDEN), jnp.float32),
        mesh=_mesh,
        scratch_types=[
            pltpu.VMEM((CHUNKS_PER_W, CHUNK), jnp.int32),
            pltpu.VMEM((CHUNKS_PER_W, CHUNK), jnp.int32),
            pltpu.VMEM((CHUNK, HIDDEN), jnp.float32),
            pltpu.VMEM_SHARED((ACC_ROWS, HIDDEN), jnp.float32),
            pltpu.SemaphoreType.DMA,
        ],
    )
    def k(g_h, src_h, dst_h, out_h, src_v, dst_v, rows_v, acc, sem):
        c = lax.axis_index("c")
        s = lax.axis_index("s")
        wid = c * NS + s
        base = s * ROWS_PER_SUB

        pltpu.sync_copy(src_h.at[wid], src_v)
        pltpu.sync_copy(dst_h.at[wid], dst_v)

        _fill(rows_v, 0.0)

        @pl.loop(0, ZCHUNKS)
        def _(i):
            pltpu.sync_copy(rows_v, acc.at[pl.ds(base + i * CHUNK, CHUNK)])

        plsc.subcore_barrier()

        @pl.loop(0, CHUNKS_PER_W)
        def _(j):
            pltpu.async_copy(g_h.at[src_v.at[j]], rows_v, sem).wait()
            pltpu.sync_copy(rows_v, acc.at[dst_v.at[j]], add=True)

        plsc.subcore_barrier()
        pltpu.sync_copy(acc.at[pl.ds(base, ROWS_PER_SUB)],
                        out_h.at[c, pl.ds(base, ROWS_PER_SUB)])

    return k(g, src_hbm, dst_hbm)


_ROWS_BLK = 1000
_GRID = N_NODES // _ROWS_BLK


def _tc_matmul(x, w):
    def body(x_ref, w_ref, o_ref):
        o_ref[...] = jnp.dot(x_ref[...], w_ref[...],
                             preferred_element_type=jnp.float32)

    return pl.pallas_call(
        body,
        grid=(_GRID,),
        in_specs=[
            pl.BlockSpec((_ROWS_BLK, D_FEAT), lambda i: (i, 0)),
            pl.BlockSpec((D_FEAT, HIDDEN), lambda i: (0, 0)),
        ],
        out_specs=pl.BlockSpec((_ROWS_BLK, HIDDEN), lambda i: (i, 0)),
        out_shape=jax.ShapeDtypeStruct((N_NODES, HIDDEN), jnp.float32),
    )(x, w)


def _dinv_from(deg_ref):
    deg = deg_ref[0][:, 0:1] + deg_ref[1][:, 0:1] + 1.0
    return lax.rsqrt(deg)


def _tc_scale(h, deg):
    def body(h_ref, deg_ref, o_ref):
        o_ref[...] = h_ref[...] * _dinv_from(deg_ref)

    return pl.pallas_call(
        body,
        grid=(_GRID,),
        in_specs=[
            pl.BlockSpec((_ROWS_BLK, HIDDEN), lambda i: (i, 0)),
            pl.BlockSpec((NC, _ROWS_BLK, LANES), lambda i: (0, i, 0)),
        ],
        out_specs=pl.BlockSpec((_ROWS_BLK, HIDDEN), lambda i: (i, 0)),
        out_shape=jax.ShapeDtypeStruct((N_NODES, HIDDEN), jnp.float32),
    )(h, deg)


def _tc_combine_matmul(s_parts, g, deg, b, w):
    def body(s_ref, g_ref, deg_ref, b_ref, w_ref, o_ref):
        dinv = _dinv_from(deg_ref)
        out = jnp.maximum(
            dinv * (s_ref[0] + s_ref[1] + g_ref[...]) + b_ref[...], 0.0)
        o_ref[...] = jnp.dot(out, w_ref[...],
                             preferred_element_type=jnp.float32) * dinv

    return pl.pallas_call(
        body,
        grid=(_GRID,),
        in_specs=[
            pl.BlockSpec((NC, _ROWS_BLK, HIDDEN), lambda i: (0, i, 0)),
            pl.BlockSpec((_ROWS_BLK, HIDDEN), lambda i: (i, 0)),
            pl.BlockSpec((NC, _ROWS_BLK, LANES), lambda i: (0, i, 0)),
            pl.BlockSpec((1, HIDDEN), lambda i: (0, 0)),
            pl.BlockSpec((HIDDEN, HIDDEN), lambda i: (0, 0)),
        ],
        out_specs=pl.BlockSpec((_ROWS_BLK, HIDDEN), lambda i: (i, 0)),
        out_shape=jax.ShapeDtypeStruct((N_NODES, HIDDEN), jnp.float32),
    )(s_parts, g, deg, b, w)


def _tc_combine(s_parts, g, deg, b):
    def body(s_ref, g_ref, deg_ref, b_ref, o_ref):
        dinv = _dinv_from(deg_ref)
        o_ref[...] = jnp.maximum(
            dinv * (s_ref[0] + s_ref[1] + g_ref[...]) + b_ref[...], 0.0)

    return pl.pallas_call(
        body,
        grid=(_GRID,),
        in_specs=[
            pl.BlockSpec((NC, _ROWS_BLK, HIDDEN), lambda i: (0, i, 0)),
            pl.BlockSpec((_ROWS_BLK, HIDDEN), lambda i: (i, 0)),
            pl.BlockSpec((NC, _ROWS_BLK, LANES), lambda i: (0, i, 0)),
            pl.BlockSpec((1, HIDDEN), lambda i: (0, 0)),
        ],
        out_specs=pl.BlockSpec((_ROWS_BLK, HIDDEN), lambda i: (i, 0)),
        out_shape=jax.ShapeDtypeStruct((N_NODES, HIDDEN), jnp.float32),
    )(s_parts, g, deg, b)


def kernel(x, edge_index, W1, b1, W2, b2):
    src = edge_index[0].astype(jnp.int32)
    dst = edge_index[1].astype(jnp.int32)

    pad = E_PAD - N_EDGES
    src_p = jnp.concatenate(
        [src, jnp.zeros((pad,), jnp.int32)]).reshape(NW, CHUNKS_PER_W, CHUNK)
    dst_p = jnp.concatenate(
        [dst, jnp.full((pad,), TRASH, jnp.int32)]).reshape(NW, CHUNKS_PER_W, CHUNK)

    b1r = b1.reshape(1, HIDDEN)
    b2r = b2.reshape(1, HIDDEN)

    deg = _sc_degree(dst_p)
    h1 = _tc_matmul(x, W1)

    g1 = _tc_scale(h1, deg)
    s1 = _sc_segment_sum(g1, src_p, dst_p)
    g2 = _tc_combine_matmul(s1, g1, deg, b1r, W2)
    s2 = _sc_segment_sum(g2, src_p, dst_p)
    return _tc_combine(s2, g2, deg, b2r)


# speedup vs baseline: 10.4345x; 1.2031x over previous
import functools

import jax
import jax.numpy as jnp
from jax import lax
from jax.experimental import pallas as pl
from jax.experimental.pallas import tpu as pltpu
from jax.experimental.pallas import tpu_sc as plsc

N_NODES = 10000
D_FEAT = 128
N_EDGES = 320000
HIDDEN = 128

NC = 2
NS = 16
NW = NC * NS
LANES = 16

CHUNK = 128
CHUNKS_PER_W = 80
EDGES_PER_W = CHUNK * CHUNKS_PER_W
E_PAD = EDGES_PER_W * NW
ACC_ROWS = 10240
ROWS_PER_SUB = ACC_ROWS // NS
ZCHUNKS = ROWS_PER_SUB // CHUNK
TRASH = N_NODES

_mesh = plsc.VectorSubcoreMesh(core_axis_name="c", subcore_axis_name="s")


def _fill(ref, value):
    rows, cols = ref.shape

    @pl.loop(0, rows)
    def _(i):
        @pl.loop(0, cols, step=LANES)
        def _(j):
            ref[i, pl.ds(j, LANES)] = jnp.full((LANES,), value, jnp.float32)


def _sc_degree(dst_hbm):
    @functools.partial(
        pl.kernel,
        out_type=jax.ShapeDtypeStruct((NC, ACC_ROWS, LANES), jnp.float32),
        mesh=_mesh,
        scratch_types=[
            pltpu.VMEM((CHUNKS_PER_W, CHUNK), jnp.int32),
            pltpu.VMEM((CHUNK, LANES), jnp.float32),
            pltpu.VMEM_SHARED((ACC_ROWS, LANES), jnp.float32),
        ],
    )
    def k(dst_h, out_h, dst_v, const_v, acc):
        c = lax.axis_index("c")
        s = lax.axis_index("s")
        wid = c * NS + s
        base = s * ROWS_PER_SUB

        pltpu.sync_copy(dst_h.at[wid, pl.ds(0, CHUNKS_PER_W)], dst_v)

        _fill(const_v, 0.0)

        @pl.loop(0, ZCHUNKS)
        def _(i):
            pltpu.sync_copy(const_v, acc.at[pl.ds(base + i * CHUNK, CHUNK)])

        _fill(const_v, 1.0)
        plsc.subcore_barrier()

        @pl.loop(0, CHUNKS_PER_W)
        def _(j):
            pltpu.sync_copy(const_v, acc.at[dst_v.at[j]], add=True)

        plsc.subcore_barrier()
        pltpu.sync_copy(acc.at[pl.ds(base, ROWS_PER_SUB)],
                        out_h.at[c, pl.ds(base, ROWS_PER_SUB)])

    return k(dst_hbm)


RB = 8
N_ROUNDS = CHUNKS_PER_W // RB
IDX_CHUNKS = (N_ROUNDS + 2) * RB


def _sc_segment_sum(g, src_hbm, dst_hbm):
    @functools.partial(
        pl.kernel,
        out_type=jax.ShapeDtypeStruct((NC, ACC_ROWS, HIDDEN), jnp.float32),
        mesh=_mesh,
        scratch_types=[pltpu.VMEM((RB, CHUNK), jnp.int32)] * 4
        + [pltpu.VMEM((CHUNK, HIDDEN), jnp.float32)] * 2
        + [pltpu.VMEM_SHARED((ACC_ROWS, HIDDEN), jnp.float32)]
        + [pltpu.SemaphoreType.DMA] * 6,
    )
    def k(g_h, src_h, dst_h, out_h,
          srcA, dstA, srcB, dstB, rows0, rows1, acc,
          gsem0, gsem1, ssem0, ssem1, isemA, isemB):
        rows = (rows0, rows1)
        gsem = (gsem0, gsem1)
        ssem = (ssem0, ssem1)
        c = lax.axis_index("c")
        s = lax.axis_index("s")
        wid = c * NS + s
        base = s * ROWS_PER_SUB

        def fire_idx(r, sv, dv, sem):
            pltpu.async_copy(src_h.at[wid, pl.ds(r * RB, RB)], sv, sem)
            pltpu.async_copy(dst_h.at[wid, pl.ds(r * RB, RB)], dv, sem)

        def wait_idx(sv, dv, sem):
            pltpu.make_async_copy(src_h.at[wid, pl.ds(0, RB)], sv, sem).wait()
            pltpu.make_async_copy(dst_h.at[wid, pl.ds(0, RB)], dv, sem).wait()

        def do_round(sv, dv):
            gc = [None] * RB
            sc = [None] * RB
            gc[0] = pltpu.async_copy(g_h.at[sv.at[0]], rows[0], gsem[0])
            for kk in range(RB):
                b = kk & 1
                nb = 1 - b
                if kk + 1 < RB:
                    if kk >= 1:
                        sc[kk - 1].wait()
                    gc[kk + 1] = pltpu.async_copy(
                        g_h.at[sv.at[kk + 1]], rows[nb], gsem[nb])
                gc[kk].wait()
                sc[kk] = pltpu.async_copy(
                    rows[b], acc.at[dv.at[kk]], ssem[b], add=True)
            sc[RB - 2].wait()
            sc[RB - 1].wait()

        fire_idx(0, srcA, dstA, isemA)
        fire_idx(1, srcB, dstB, isemB)

        _fill(rows0, 0.0)

        @pl.loop(0, ZCHUNKS)
        def _(i):
            pltpu.sync_copy(rows0, acc.at[pl.ds(base + i * CHUNK, CHUNK)])

        plsc.subcore_barrier()

        @pl.loop(0, N_ROUNDS // 2)
        def _(j):
            rA = 2 * j
            wait_idx(srcA, dstA, isemA)
            do_round(srcA, dstA)
            fire_idx(rA + 2, srcA, dstA, isemA)
            wait_idx(srcB, dstB, isemB)
            do_round(srcB, dstB)
            fire_idx(rA + 3, srcB, dstB, isemB)

        wait_idx(srcA, dstA, isemA)
        wait_idx(srcB, dstB, isemB)

        plsc.subcore_barrier()
        pltpu.sync_copy(acc.at[pl.ds(base, ROWS_PER_SUB)],
                        out_h.at[c, pl.ds(base, ROWS_PER_SUB)])

    return k(g, src_hbm, dst_hbm)


_ROWS_BLK = 1000
_GRID = N_NODES // _ROWS_BLK


def _tc_matmul(x, w):
    def body(x_ref, w_ref, o_ref):
        o_ref[...] = jnp.dot(x_ref[...], w_ref[...],
                             preferred_element_type=jnp.float32)

    return pl.pallas_call(
        body,
        grid=(_GRID,),
        in_specs=[
            pl.BlockSpec((_ROWS_BLK, D_FEAT), lambda i: (i, 0)),
            pl.BlockSpec((D_FEAT, HIDDEN), lambda i: (0, 0)),
        ],
        out_specs=pl.BlockSpec((_ROWS_BLK, HIDDEN), lambda i: (i, 0)),
        out_shape=jax.ShapeDtypeStruct((N_NODES, HIDDEN), jnp.float32),
    )(x, w)


def _dinv_from(deg_ref):
    deg = deg_ref[0][:, 0:1] + deg_ref[1][:, 0:1] + 1.0
    return lax.rsqrt(deg)


def _tc_scale(h, deg):
    def body(h_ref, deg_ref, o_ref):
        o_ref[...] = h_ref[...] * _dinv_from(deg_ref)

    return pl.pallas_call(
        body,
        grid=(_GRID,),
        in_specs=[
            pl.BlockSpec((_ROWS_BLK, HIDDEN), lambda i: (i, 0)),
            pl.BlockSpec((NC, _ROWS_BLK, LANES), lambda i: (0, i, 0)),
        ],
        out_specs=pl.BlockSpec((_ROWS_BLK, HIDDEN), lambda i: (i, 0)),
        out_shape=jax.ShapeDtypeStruct((N_NODES, HIDDEN), jnp.float32),
    )(h, deg)


def _tc_combine_matmul(s_parts, g, deg, b, w):
    def body(s_ref, g_ref, deg_ref, b_ref, w_ref, o_ref):
        dinv = _dinv_from(deg_ref)
        out = jnp.maximum(
            dinv * (s_ref[0] + s_ref[1] + g_ref[...]) + b_ref[...], 0.0)
        o_ref[...] = jnp.dot(out, w_ref[...],
                             preferred_element_type=jnp.float32) * dinv

    return pl.pallas_call(
        body,
        grid=(_GRID,),
        in_specs=[
            pl.BlockSpec((NC, _ROWS_BLK, HIDDEN), lambda i: (0, i, 0)),
            pl.BlockSpec((_ROWS_BLK, HIDDEN), lambda i: (i, 0)),
            pl.BlockSpec((NC, _ROWS_BLK, LANES), lambda i: (0, i, 0)),
            pl.BlockSpec((1, HIDDEN), lambda i: (0, 0)),
            pl.BlockSpec((HIDDEN, HIDDEN), lambda i: (0, 0)),
        ],
        out_specs=pl.BlockSpec((_ROWS_BLK, HIDDEN), lambda i: (i, 0)),
        out_shape=jax.ShapeDtypeStruct((N_NODES, HIDDEN), jnp.float32),
    )(s_parts, g, deg, b, w)


def _tc_combine(s_parts, g, deg, b):
    def body(s_ref, g_ref, deg_ref, b_ref, o_ref):
        dinv = _dinv_from(deg_ref)
        o_ref[...] = jnp.maximum(
            dinv * (s_ref[0] + s_ref[1] + g_ref[...]) + b_ref[...], 0.0)

    return pl.pallas_call(
        body,
        grid=(_GRID,),
        in_specs=[
            pl.BlockSpec((NC, _ROWS_BLK, HIDDEN), lambda i: (0, i, 0)),
            pl.BlockSpec((_ROWS_BLK, HIDDEN), lambda i: (i, 0)),
            pl.BlockSpec((NC, _ROWS_BLK, LANES), lambda i: (0, i, 0)),
            pl.BlockSpec((1, HIDDEN), lambda i: (0, 0)),
        ],
        out_specs=pl.BlockSpec((_ROWS_BLK, HIDDEN), lambda i: (i, 0)),
        out_shape=jax.ShapeDtypeStruct((N_NODES, HIDDEN), jnp.float32),
    )(s_parts, g, deg, b)


def kernel(x, edge_index, W1, b1, W2, b2):
    src = edge_index[0].astype(jnp.int32)
    dst = edge_index[1].astype(jnp.int32)

    pad = E_PAD - N_EDGES
    src_p = jnp.concatenate(
        [src, jnp.zeros((pad,), jnp.int32)]).reshape(NW, CHUNKS_PER_W, CHUNK)
    dst_p = jnp.concatenate(
        [dst, jnp.full((pad,), TRASH, jnp.int32)]).reshape(NW, CHUNKS_PER_W, CHUNK)
    dummy = jnp.zeros((NW, IDX_CHUNKS - CHUNKS_PER_W, CHUNK), jnp.int32)
    src_p = jnp.concatenate([src_p, dummy], axis=1)
    dst_p = jnp.concatenate([dst_p, dummy], axis=1)

    b1r = b1.reshape(1, HIDDEN)
    b2r = b2.reshape(1, HIDDEN)

    deg = _sc_degree(dst_p)
    h1 = _tc_matmul(x, W1)

    g1 = _tc_scale(h1, deg)
    s1 = _sc_segment_sum(g1, src_p, dst_p)
    g2 = _tc_combine_matmul(s1, g1, deg, b1r, W2)
    s2 = _sc_segment_sum(g2, src_p, dst_p)
    return _tc_combine(s2, g2, deg, b2r)
